# baseline (speedup 1.0000x reference)
import functools

import jax
import jax.numpy as jnp
from jax import lax
from jax.experimental import pallas as pl
from jax.experimental.pallas import tpu as pltpu
from jax.experimental.pallas import tpu_sc as plsc

B, S, D_MODEL = 1, 2048, 768
RANK = 32
N_COMPRESS = 64
TOP_K = 2

BLK = 256
NEG = -1e30
NW = 32
TPW = S // NW
L = 16


def _tc_body(x_ref, wr_ref, wf_ref, scores_ref, proj_ref):
    x_blk = x_ref[...]
    scores_ref[...] = jnp.dot(x_blk, wr_ref[...],
                              preferred_element_type=jnp.float32)
    proj = jnp.dot(x_blk.astype(jnp.bfloat16), wf_ref[...],
                   preferred_element_type=jnp.float32)
    proj_ref[...] = proj.reshape(BLK * (N_COMPRESS // 4), 4 * RANK)


def _tc_stage(x2d, wr_t, wf):
    return pl.pallas_call(
        _tc_body,
        grid=(S // BLK,),
        in_specs=[
            pl.BlockSpec((BLK, D_MODEL), lambda i: (i, 0)),
            pl.BlockSpec((D_MODEL, N_COMPRESS), lambda i: (0, 0)),
            pl.BlockSpec((D_MODEL, N_COMPRESS * RANK), lambda i: (0, 0)),
        ],
        out_specs=[
            pl.BlockSpec((BLK, N_COMPRESS), lambda i: (i, 0)),
            pl.BlockSpec((BLK * (N_COMPRESS // 4), 4 * RANK), lambda i: (i, 0)),
        ],
        out_shape=[
            jax.ShapeDtypeStruct((S, N_COMPRESS), jnp.float32),
            jax.ShapeDtypeStruct((S * (N_COMPRESS // 4), 4 * RANK), jnp.float32),
        ],
    )(x2d, wr_t, wf)


def _sc_body(scores_hbm, proj_hbm, out_hbm, w_hbm, idx_hbm,
             score_v, idx1_v, idx2_v, rows1_v, rows2_v,
             out_v, w_v, ti_v, sem):
    wid = lax.axis_index("s") * 2 + lax.axis_index("c")
    base = wid * TPW
    pltpu.sync_copy(scores_hbm.at[pl.ds(base, TPW)], score_v)

    lanes = lax.iota(jnp.int32, L)
    zero_f = jnp.zeros((L,), jnp.float32)
    zero_i = jnp.zeros((L,), jnp.int32)

    for c in range(TPW // L):
        tok = c * L + lanes

        m1, i1, m2, i2 = zero_f + NEG, zero_i, zero_f + NEG, zero_i
        for n in range(N_COMPRESS):
            col = jnp.full((L,), n, jnp.int32)
            v = plsc.load_gather(score_v, [tok, col])
            gt1 = v > m1
            gt2 = jnp.logical_and(jnp.logical_not(gt1), v > m2)
            m2 = jnp.where(gt1, m1, jnp.where(gt2, v, m2))
            i2 = jnp.where(gt1, i1, jnp.where(gt2, col, i2))
            m1 = jnp.where(gt1, v, m1)
            i1 = jnp.where(gt1, col, i1)

        e = jnp.exp(m2 - m1)
        w1 = 1.0 / (1.0 + e)
        w2 = 1.0 - w1

        plsc.store_scatter(w_v, [tok, zero_i], w1)
        plsc.store_scatter(w_v, [tok, zero_i + 1], w2)
        plsc.store_scatter(ti_v, [tok, zero_i], i1)
        plsc.store_scatter(ti_v, [tok, zero_i + 1], i2)

        g1 = (base + tok) * (N_COMPRESS // 4) + (i1 >> 2)
        g2 = (base + tok) * (N_COMPRESS // 4) + (i2 >> 2)
        idx1_v[pl.ds(c * L, L)] = g1
        idx2_v[pl.ds(c * L, L)] = g2

    pltpu.async_copy(proj_hbm.at[idx1_v], rows1_v, sem).wait()
    pltpu.async_copy(proj_hbm.at[idx2_v], rows2_v, sem).wait()

    for c in range(TPW // L):
        tok = c * L + lanes
        w1 = plsc.load_gather(w_v, [tok, zero_i])
        w2 = plsc.load_gather(w_v, [tok, zero_i + 1])
        i1 = plsc.load_gather(ti_v, [tok, zero_i])
        i2 = plsc.load_gather(ti_v, [tok, zero_i + 1])
        cb1 = (i1 & 3) * RANK
        cb2 = (i2 & 3) * RANK
        for r in range(RANK):
            col = jnp.full((L,), r, jnp.int32)
            v1 = plsc.load_gather(rows1_v, [tok, cb1 + r])
            v2 = plsc.load_gather(rows2_v, [tok, cb2 + r])
            plsc.store_scatter(out_v, [tok, col], w1 * v1 + w2 * v2)

    pltpu.sync_copy(out_v, out_hbm.at[pl.ds(base, TPW)])
    pltpu.sync_copy(w_v, w_hbm.at[pl.ds(base, TPW)])
    pltpu.sync_copy(ti_v, idx_hbm.at[pl.ds(base, TPW)])


def _sc_stage(scores, proj_flat):
    mesh = plsc.VectorSubcoreMesh(core_axis_name="c", subcore_axis_name="s")
    run = pl.kernel(
        _sc_body,
        mesh=mesh,
        out_type=[
            jax.ShapeDtypeStruct((S, RANK), jnp.float32),
            jax.ShapeDtypeStruct((S, TOP_K), jnp.float32),
            jax.ShapeDtypeStruct((S, TOP_K), jnp.int32),
        ],
        scratch_types=[
            pltpu.VMEM((TPW, N_COMPRESS), jnp.float32),
            pltpu.VMEM((TPW,), jnp.int32),
            pltpu.VMEM((TPW,), jnp.int32),
            pltpu.VMEM((TPW, 4 * RANK), jnp.float32),
            pltpu.VMEM((TPW, 4 * RANK), jnp.float32),
            pltpu.VMEM((TPW, RANK), jnp.float32),
            pltpu.VMEM((TPW, TOP_K), jnp.float32),
            pltpu.VMEM((TPW, TOP_K), jnp.int32),
            pltpu.SemaphoreType.DMA,
        ],
        compiler_params=pltpu.CompilerParams(needs_layout_passes=False),
    )
    return run(scores, proj_flat)


@jax.jit
def kernel(x, W_router, compress_neurons):
    x2d = x.reshape(S, D_MODEL)
    wr_t = W_router.T
    wf = compress_neurons.transpose(1, 0, 2).reshape(
        D_MODEL, N_COMPRESS * RANK).astype(jnp.bfloat16)
    scores, proj_flat = _tc_stage(x2d, wr_t, wf)
    out, w, idx = _sc_stage(scores, proj_flat)
    return (out.reshape(B, S, RANK), w.reshape(B, S, TOP_K),
            idx.reshape(B, S, TOP_K))


# speedup vs baseline: 1.0397x; 1.0397x over previous
import functools

import jax
import jax.numpy as jnp
from jax import lax
from jax.experimental import pallas as pl
from jax.experimental.pallas import tpu as pltpu
from jax.experimental.pallas import tpu_sc as plsc

B, S, D_MODEL = 1, 2048, 768
RANK = 32
N_COMPRESS = 64
TOP_K = 2

BLK = 256
NEG = -1e30
NW = 32
TPW = S // NW
L = 16


def _tc_body(x_ref, wr_ref, wf_ref, scores_ref, proj_ref):
    x_blk = x_ref[...]
    scores_ref[...] = jnp.dot(x_blk, wr_ref[...],
                              preferred_element_type=jnp.float32)
    proj = jnp.dot(x_blk.astype(jnp.bfloat16), wf_ref[...],
                   preferred_element_type=jnp.float32)
    proj_ref[...] = proj.reshape(BLK * (N_COMPRESS // 4), 4 * RANK)


def _tc_stage(x2d, wr_t, wf):
    return pl.pallas_call(
        _tc_body,
        grid=(S // BLK,),
        in_specs=[
            pl.BlockSpec((BLK, D_MODEL), lambda i: (i, 0)),
            pl.BlockSpec((D_MODEL, N_COMPRESS), lambda i: (0, 0)),
            pl.BlockSpec((D_MODEL, N_COMPRESS * RANK), lambda i: (0, 0)),
        ],
        out_specs=[
            pl.BlockSpec((BLK, N_COMPRESS), lambda i: (i, 0)),
            pl.BlockSpec((BLK * (N_COMPRESS // 4), 4 * RANK), lambda i: (i, 0)),
        ],
        out_shape=[
            jax.ShapeDtypeStruct((S, N_COMPRESS), jnp.float32),
            jax.ShapeDtypeStruct((S * (N_COMPRESS // 4), 4 * RANK), jnp.float32),
        ],
    )(x2d, wr_t, wf)


def _sc_body(scores_hbm, proj_hbm, out_hbm, w_hbm, idx_hbm,
             score_v, idx1_v, idx2_v, rows1_v, rows2_v,
             out_v, w_v, ti_v, sem):
    wid = lax.axis_index("s") * 2 + lax.axis_index("c")
    base = wid * TPW
    pltpu.sync_copy(scores_hbm.at[pl.ds(base, TPW)], score_v)

    lanes = lax.iota(jnp.int32, L)
    zero_f = jnp.zeros((L,), jnp.float32)
    zero_i = jnp.zeros((L,), jnp.int32)

    for c in range(TPW // L):
        tok = c * L + lanes

        def scan_body(nb, carry):
            m1, i1, m2, i2 = carry
            for j in range(4):
                col = nb * 4 + j + jnp.zeros((L,), jnp.int32)
                v = plsc.load_gather(score_v, [tok, col])
                gt1 = v > m1
                gt2 = jnp.logical_and(jnp.logical_not(gt1), v > m2)
                m2 = jnp.where(gt1, m1, jnp.where(gt2, v, m2))
                i2 = jnp.where(gt1, i1, jnp.where(gt2, col, i2))
                m1 = jnp.where(gt1, v, m1)
                i1 = jnp.where(gt1, col, i1)
            return (m1, i1, m2, i2)

        m1, i1, m2, i2 = lax.fori_loop(
            0, N_COMPRESS // 4, scan_body,
            (zero_f + NEG, zero_i, zero_f + NEG, zero_i))

        e = jnp.exp(m2 - m1)
        w1 = 1.0 / (1.0 + e)
        w2 = 1.0 - w1

        plsc.store_scatter(w_v, [tok, zero_i], w1)
        plsc.store_scatter(w_v, [tok, zero_i + 1], w2)
        plsc.store_scatter(ti_v, [tok, zero_i], i1)
        plsc.store_scatter(ti_v, [tok, zero_i + 1], i2)

        g1 = (base + tok) * (N_COMPRESS // 4) + (i1 >> 2)
        g2 = (base + tok) * (N_COMPRESS // 4) + (i2 >> 2)
        idx1_v[pl.ds(c * L, L)] = g1
        idx2_v[pl.ds(c * L, L)] = g2

    pltpu.async_copy(proj_hbm.at[idx1_v], rows1_v, sem).wait()
    pltpu.async_copy(proj_hbm.at[idx2_v], rows2_v, sem).wait()

    for c in range(TPW // L):
        tok = c * L + lanes
        w1 = plsc.load_gather(w_v, [tok, zero_i])
        w2 = plsc.load_gather(w_v, [tok, zero_i + 1])
        i1 = plsc.load_gather(ti_v, [tok, zero_i])
        i2 = plsc.load_gather(ti_v, [tok, zero_i + 1])
        cb1 = (i1 & 3) * RANK
        cb2 = (i2 & 3) * RANK
        for r in range(RANK):
            col = jnp.full((L,), r, jnp.int32)
            v1 = plsc.load_gather(rows1_v, [tok, cb1 + r])
            v2 = plsc.load_gather(rows2_v, [tok, cb2 + r])
            plsc.store_scatter(out_v, [tok, col], w1 * v1 + w2 * v2)

    pltpu.sync_copy(out_v, out_hbm.at[0, pl.ds(base, TPW)])
    pltpu.sync_copy(w_v, w_hbm.at[0, pl.ds(base, TPW)])
    pltpu.sync_copy(ti_v, idx_hbm.at[0, pl.ds(base, TPW)])


def _sc_stage(scores, proj_flat):
    mesh = plsc.VectorSubcoreMesh(core_axis_name="c", subcore_axis_name="s")
    run = pl.kernel(
        _sc_body,
        mesh=mesh,
        out_type=[
            jax.ShapeDtypeStruct((B, S, RANK), jnp.float32),
            jax.ShapeDtypeStruct((B, S, TOP_K), jnp.float32),
            jax.ShapeDtypeStruct((B, S, TOP_K), jnp.int32),
        ],
        scratch_types=[
            pltpu.VMEM((TPW, N_COMPRESS), jnp.float32),
            pltpu.VMEM((TPW,), jnp.int32),
            pltpu.VMEM((TPW,), jnp.int32),
            pltpu.VMEM((TPW, 4 * RANK), jnp.float32),
            pltpu.VMEM((TPW, 4 * RANK), jnp.float32),
            pltpu.VMEM((TPW, RANK), jnp.float32),
            pltpu.VMEM((TPW, TOP_K), jnp.float32),
            pltpu.VMEM((TPW, TOP_K), jnp.int32),
            pltpu.SemaphoreType.DMA,
        ],
        compiler_params=pltpu.CompilerParams(needs_layout_passes=False),
    )
    return run(scores, proj_flat)


@jax.jit
def kernel(x, W_router, compress_neurons):
    x2d = x.reshape(S, D_MODEL)
    wr_t = W_router.T
    wf = compress_neurons.astype(jnp.bfloat16).transpose(1, 0, 2).reshape(
        D_MODEL, N_COMPRESS * RANK)
    scores, proj_flat = _tc_stage(x2d, wr_t, wf)
    return _sc_stage(scores, proj_flat)
